# Initial kernel scaffold; baseline (speedup 1.0000x reference)
import jax
import jax.numpy as jnp
from jax.experimental import pallas as pl


def kernel(center_words, context_words, neg_words, in_embed, out_embed):
    raise NotImplementedError("write your pallas kernel here")


# speedup vs baseline: 3.6061x; 3.6061x over previous
import functools

import jax
import jax.numpy as jnp
from jax import lax
from jax.experimental import pallas as pl
from jax.experimental.pallas import tpu as pltpu
from jax.experimental.pallas import tpu_sc as plsc

NC = 2
NS = 16
LANES = 16
NW = NC * NS

DIM = 64
NEGS = 20
SB = 128
GROUPS = SB // LANES
DUNROLL = 8


def _iota16():
    return lax.iota(jnp.int32, LANES)


def _log_sigmoid(x):
    u = jnp.exp(-jnp.abs(x))
    z = u / (u + 2.0)
    z2 = z * z
    p = 1.0 + z2 * (1.0 / 3.0 + z2 * (0.2 + z2 * (1.0 / 7.0)))
    return jnp.minimum(x, 0.0) - 2.0 * z * p


def _gather16(ref, rows, cols):
    return plsc.load_gather(ref, [rows, cols])


def _dot_group(a_ref, a_rows, b_ref, b_rows):
    zero = jnp.zeros((LANES,), jnp.float32)

    def body(i, accs):
        d0 = i * DUNROLL
        out = []
        for j in range(DUNROLL):
            col = jnp.full((LANES,), d0 + j, jnp.int32)
            av = _gather16(a_ref, a_rows, col)
            bv = _gather16(b_ref, b_rows, col)
            out.append(accs[j] + av * bv)
        return tuple(out)

    accs = lax.fori_loop(0, DIM // DUNROLL, body, (zero,) * DUNROLL)
    r = accs[0]
    for a in accs[1:]:
        r = r + a
    return r


def _gather_rows(table_r, idx_view, dst, sem):
    return pltpu.async_copy(table_r.at[idx_view], dst, sem)


def _make_sc_call(batch):
    rpw = batch // NW
    nsb = rpw // SB

    def body(center_r, context_r, negflat_r, in_emb_r, out_emb_r, out_r,
             idx_ctr, idx_pos, idx_neg, ctr_buf, pos_buf, neg_buf, stage, sem):
        wid = lax.axis_index("s") * NC + lax.axis_index("c")
        base = pl.multiple_of(wid * rpw, 8)
        pltpu.sync_copy(center_r.at[pl.ds(base, rpw)], idx_ctr)
        pltpu.sync_copy(context_r.at[pl.ds(base, rpw)], idx_pos)

        def sbody(sb, tot):
            off = pl.multiple_of(sb * SB, 8)
            pltpu.sync_copy(
                negflat_r.at[pl.ds((base + off) * NEGS, SB * NEGS)], idx_neg)
            c1 = _gather_rows(in_emb_r, idx_ctr.at[pl.ds(off, SB)], ctr_buf, sem)
            c2 = _gather_rows(out_emb_r, idx_pos.at[pl.ds(off, SB)], pos_buf, sem)
            c1.wait()
            c2.wait()
            for g in range(GROUPS):
                rows = g * LANES + _iota16()
                s = _dot_group(ctr_buf, rows, pos_buf, rows)
                tot = tot + _log_sigmoid(s)

            def kbody(k, t):
                koff = pl.multiple_of(k * SB, 8)
                _gather_rows(out_emb_r, idx_neg.at[pl.ds(koff, SB)], neg_buf,
                             sem).wait()
                for g in range(GROUPS):
                    rowsn = g * LANES + _iota16()
                    flat = koff + rowsn
                    rowsc = flat // NEGS
                    s = _dot_group(ctr_buf, rowsc, neg_buf, rowsn)
                    t = t + _log_sigmoid(-s)
                return t

            return lax.fori_loop(0, NEGS, kbody, tot)

        tot = lax.fori_loop(0, nsb, sbody, jnp.zeros((LANES,), jnp.float32))
        stage[...] = tot * (-1.0 / batch)
        pltpu.sync_copy(stage, out_r.at[wid])

    mesh = plsc.VectorSubcoreMesh(
        core_axis_name="c", subcore_axis_name="s",
        num_cores=NC, num_subcores=NS)
    return pl.kernel(
        body,
        out_type=jax.ShapeDtypeStruct((NW, LANES), jnp.float32),
        mesh=mesh,
        compiler_params=pltpu.CompilerParams(
            needs_layout_passes=False, use_tc_tiling_on_sc=False),
        scratch_types=[
            pltpu.VMEM((rpw,), jnp.int32),
            pltpu.VMEM((rpw,), jnp.int32),
            pltpu.VMEM((SB * NEGS,), jnp.int32),
            pltpu.VMEM((SB, DIM), jnp.float32),
            pltpu.VMEM((SB, DIM), jnp.float32),
            pltpu.VMEM((SB, DIM), jnp.float32),
            pltpu.VMEM((LANES,), jnp.float32),
            pltpu.SemaphoreType.DMA,
        ],
    )


@jax.jit
def kernel(center_words, context_words, neg_words, in_embed, out_embed):
    batch = center_words.shape[0]
    call = _make_sc_call(batch)
    partials = call(
        center_words.astype(jnp.int32),
        context_words.astype(jnp.int32),
        neg_words.reshape(-1).astype(jnp.int32),
        in_embed,
        out_embed,
    )
    return jnp.sum(partials)
